# baseline (speedup 1.0000x reference)
import functools

import jax
import jax.numpy as jnp
from jax import lax
from jax.experimental import pallas as pl
from jax.experimental.pallas import tpu as pltpu
from jax.experimental.pallas import tpu_sc as plsc

SEG = 8
NB = 8
F = NB * SEG
C = 512
FOLD = 128
HW = 784
G = 2
CHUNK = 64
NC = 2
NS = 16
NW = NC * NS
FRAMES_PER_W = F // NW


def _pool_body(x_ref, o_ref):
    o_ref[0, 0] = jnp.sum(x_ref[0, 0], axis=-1) * (1.0 / HW)


def _pool(x4):
    return pl.pallas_call(
        _pool_body,
        grid=(F,),
        in_specs=[pl.BlockSpec((1, 1, FOLD, HW), lambda i: (i, 0, 0, 0))],
        out_specs=pl.BlockSpec((1, 1, FOLD), lambda i: (i, 0, 0)),
        out_shape=jax.ShapeDtypeStruct((F, 1, FOLD), jnp.float32),
    )(x4)


def _sigmoid(v):
    return 1.0 / (1.0 + jnp.exp(-v))


def _nets_body(p_ref, ocw_ref, ocb_ref, f1w_ref, f1b_ref, f2w_ref, f2b_ref,
               wcw_ref, wcb_ref, idx0_ref, idx1_ref, sc_ref):
    p = p_ref[...]
    zero = jnp.zeros((NB, 1, FOLD), jnp.float32)
    p_m1 = jnp.concatenate([zero, p[:, : SEG - 1]], axis=1)
    p_p1 = jnp.concatenate([p[:, 1:], zero], axis=1)

    ocw = ocw_ref[...]
    t0 = (jnp.sum(p_m1 * ocw[0][None, None, :], axis=-1)
          + jnp.sum(p * ocw[1][None, None, :], axis=-1)
          + jnp.sum(p_p1 * ocw[2][None, None, :], axis=-1)
          + ocb_ref[0, 0])
    t1 = jnp.maximum(
        jnp.dot(t0, f1w_ref[...], preferred_element_type=jnp.float32)
        + f1b_ref[0][None, :], 0.0)
    t2 = (jnp.dot(t1, f2w_ref[...], preferred_element_type=jnp.float32)
          + f2b_ref[0][None, :])
    x_offset = 4.0 * (_sigmoid(t2) - 0.5)
    off_bs = jnp.tile(x_offset, (1, SEG // G))

    off0f = jnp.floor(off_bs)
    o0 = jnp.clip(off0f.astype(jnp.int32), 0, SEG - 1)
    o1 = jnp.clip(off0f.astype(jnp.int32) + 1, 0, SEG - 1)
    w1 = off_bs - off0f
    w0 = 1.0 - w1

    wcw = wcw_ref[...]
    xw = []
    for g in range(G):
        ug = (jnp.sum(p_m1 * wcw[0, :, g][None, None, :], axis=-1)
              + jnp.sum(p * wcw[1, :, g][None, None, :], axis=-1)
              + jnp.sum(p_p1 * wcw[2, :, g][None, None, :], axis=-1)
              + wcb_ref[0, g])
        xw.append(2.0 * _sigmoid(ug))

    b_idx = lax.broadcasted_iota(jnp.int32, (NB, SEG), 0)
    base0 = ((b_idx * SEG + o0) * (C // 8)).reshape(F, 1, 1)
    base1 = ((b_idx * SEG + o1) * (C // 8)).reshape(F, 1, 1)
    ch = (lax.broadcasted_iota(jnp.int32, (1, FOLD // CHUNK, 8), 1)
          * (CHUNK // 8)
          + lax.broadcasted_iota(jnp.int32, (1, FOLD // CHUNK, 8), 2))
    idx0_ref[...] = base0 + ch
    idx1_ref[...] = base1 + ch

    s0 = jnp.stack([w0 * xw[0], w0 * xw[1]], axis=-1).reshape(F, G)
    s1 = jnp.stack([w1 * xw[0], w1 * xw[1]], axis=-1).reshape(F, G)
    sc = jnp.concatenate([s0, s1], axis=1)
    sc_ref[...] = jnp.broadcast_to(sc[:, :, None], (F, 2 * G, 16))


def _nets(pooled, ocw, ocb, f1w, f1b, f2w, f2b, wcw, wcb):
    return pl.pallas_call(
        _nets_body,
        out_shape=[
            jax.ShapeDtypeStruct((F, FOLD // CHUNK, 8), jnp.int32),
            jax.ShapeDtypeStruct((F, FOLD // CHUNK, 8), jnp.int32),
            jax.ShapeDtypeStruct((F, 2 * G, 16), jnp.float32),
        ],
    )(pooled, ocw, ocb, f1w, f1b, f2w, f2b, wcw, wcb)


ROWS = F * C // 8
RW = 8 * HW
CR = CHUNK // 8


def _sc_build():
    mesh = plsc.VectorSubcoreMesh(core_axis_name="c", subcore_axis_name="s")

    @functools.partial(
        pl.kernel,
        mesh=mesh,
        out_type=jax.ShapeDtypeStruct((ROWS, RW), jnp.float32),
        scratch_types=[
            pltpu.VMEM((FOLD // CHUNK, CR), jnp.int32),
            pltpu.VMEM((FOLD // CHUNK, CR), jnp.int32),
            pltpu.VMEM((2 * G, 16), jnp.float32),
            pltpu.VMEM((CR, RW), jnp.float32),
            pltpu.VMEM((CR, RW), jnp.float32),
            pltpu.SemaphoreType.DMA,
            pltpu.SemaphoreType.DMA,
        ],
    )
    def sc_main(xr, idx0, idx1, sc, out, vidx0, vidx1, vs, g0, g1, sem0, sem1):
        wid = lax.axis_index("s") * NC + lax.axis_index("c")
        for k in range(FRAMES_PER_W):
            i = wid * FRAMES_PER_W + k
            row_out = i * (C // 8)
            pltpu.sync_copy(idx0.at[i], vidx0)
            pltpu.sync_copy(idx1.at[i], vidx1)
            pltpu.sync_copy(sc.at[i], vs)
            for chunk in range(FOLD // CHUNK):
                cp0 = pltpu.async_copy(xr.at[vidx0.at[chunk]], g0, sem0)
                cp1 = pltpu.async_copy(xr.at[vidx1.at[chunk]], g1, sem1)
                cp0.wait()
                cp1.wait()
                s0v = vs[chunk]
                s1v = vs[G + chunk]

                for r in range(CR):
                    def _vec(v, _, r=r):
                        a = g0[r, pl.ds(v * 16, 16)]
                        b = g1[r, pl.ds(v * 16, 16)]
                        g0[r, pl.ds(v * 16, 16)] = s0v * a + s1v * b
                        return 0
                    lax.fori_loop(0, RW // 16, _vec, 0)

                pltpu.sync_copy(g0, out.at[pl.ds(row_out + chunk * CR, CR)])
            for pc in range((C - FOLD) // CHUNK):
                buf = g0 if pc % 2 == 0 else g1
                r = row_out + (FOLD // 8) + pc * CR
                pltpu.sync_copy(xr.at[pl.ds(r, CR)], buf)
                pltpu.sync_copy(buf, out.at[pl.ds(r, CR)])

    return sc_main


def kernel(x, off_conv_w, off_conv_b, off_fc1_w, off_fc1_b, off_fc2_w,
           off_fc2_b, w_conv_w, w_conv_b):
    x2 = x.reshape(F, C, HW)
    x4 = x2.reshape(F, C // FOLD, FOLD, HW)
    pooled = _pool(x4).reshape(NB, SEG, FOLD)
    idx0, idx1, sc = _nets(
        pooled,
        off_conv_w.reshape(3, FOLD), off_conv_b.reshape(1, 1),
        off_fc1_w, off_fc1_b.reshape(1, SEG),
        off_fc2_w, off_fc2_b.reshape(1, G),
        w_conv_w, w_conv_b.reshape(1, G),
    )
    out = _sc_build()(x2.reshape(ROWS, RW), idx0, idx1, sc)
    return out.reshape(F, C, 28, 28)


# speedup vs baseline: 1.8848x; 1.8848x over previous
import jax
import jax.numpy as jnp
from jax import lax
from jax.experimental import pallas as pl
from jax.experimental.pallas import tpu as pltpu

SEG = 8
NB = 8
F = NB * SEG
C = 512
FOLD = 128
H = 28
W = 28
G = 2
NCB = C // FOLD


def _pool_body(x_ref, o_ref):
    o_ref[0, 0] = jnp.sum(x_ref[0, 0], axis=(-2, -1)) * (1.0 / (H * W))


def _pool(x5):
    return pl.pallas_call(
        _pool_body,
        grid=(F,),
        in_specs=[pl.BlockSpec((1, 1, FOLD, H, W), lambda i: (i, 0, 0, 0, 0))],
        out_specs=pl.BlockSpec((1, 1, FOLD), lambda i: (i, 0, 0)),
        out_shape=jax.ShapeDtypeStruct((F, 1, FOLD), jnp.float32),
    )(x5)


def _sigmoid(v):
    return 1.0 / (1.0 + jnp.exp(-v))


def _nets_body(p_ref, ocw_ref, ocb_ref, f1w_ref, f1b_ref, f2w_ref, f2b_ref,
               wcw_ref, wcb_ref, n0_ref, n1_ref, s0_ref, s1_ref):
    p = p_ref[...]
    zero = jnp.zeros((NB, 1, FOLD), jnp.float32)
    p_m1 = jnp.concatenate([zero, p[:, : SEG - 1]], axis=1)
    p_p1 = jnp.concatenate([p[:, 1:], zero], axis=1)

    ocw = ocw_ref[...]
    t0 = (jnp.sum(p_m1 * ocw[0][None, None, :], axis=-1)
          + jnp.sum(p * ocw[1][None, None, :], axis=-1)
          + jnp.sum(p_p1 * ocw[2][None, None, :], axis=-1)
          + ocb_ref[0, 0])
    t1 = jnp.maximum(
        jnp.dot(t0, f1w_ref[...], preferred_element_type=jnp.float32)
        + f1b_ref[0][None, :], 0.0)
    t2 = (jnp.dot(t1, f2w_ref[...], preferred_element_type=jnp.float32)
          + f2b_ref[0][None, :])
    x_offset = 4.0 * (_sigmoid(t2) - 0.5)
    off_bs = jnp.tile(x_offset, (1, SEG // G))

    off0f = jnp.floor(off_bs)
    o0 = jnp.clip(off0f.astype(jnp.int32), 0, SEG - 1)
    o1 = jnp.clip(off0f.astype(jnp.int32) + 1, 0, SEG - 1)
    w1 = off_bs - off0f
    w0 = 1.0 - w1

    wcw = wcw_ref[...]
    xw = []
    for g in range(G):
        ug = (jnp.sum(p_m1 * wcw[0, :, g][None, None, :], axis=-1)
              + jnp.sum(p * wcw[1, :, g][None, None, :], axis=-1)
              + jnp.sum(p_p1 * wcw[2, :, g][None, None, :], axis=-1)
              + wcb_ref[0, g])
        xw.append(2.0 * _sigmoid(ug))

    b_idx = lax.broadcasted_iota(jnp.int32, (NB, SEG), 0)
    n0_ref[...] = b_idx * SEG + o0
    n1_ref[...] = b_idx * SEG + o1

    xw_chan = jnp.concatenate(
        [jnp.broadcast_to(xw[0][:, :, None], (NB, SEG, FOLD // G)),
         jnp.broadcast_to(xw[1][:, :, None], (NB, SEG, FOLD // G))],
        axis=-1)
    s0_ref[...] = w0[:, :, None] * xw_chan
    s1_ref[...] = w1[:, :, None] * xw_chan


def _nets(pooled, ocw, ocb, f1w, f1b, f2w, f2b, wcw, wcb):
    return pl.pallas_call(
        _nets_body,
        out_shape=[
            jax.ShapeDtypeStruct((NB, SEG), jnp.int32),
            jax.ShapeDtypeStruct((NB, SEG), jnp.int32),
            jax.ShapeDtypeStruct((NB, SEG, FOLD), jnp.float32),
            jax.ShapeDtypeStruct((NB, SEG, FOLD), jnp.float32),
        ],
    )(pooled, ocw, ocb, f1w, f1b, f2w, f2b, wcw, wcb)


def _main_body(n0_ref, n1_ref, g0_ref, g1_ref, xr_ref, s0_ref, s1_ref,
               out_ref):
    j = pl.program_id(1)

    @pl.when(j == 0)
    def _blend():
        s0 = s0_ref[0, 0, :]
        s1 = s1_ref[0, 0, :]
        out_ref[0, 0] = (s0[:, None, None] * g0_ref[0, 0]
                         + s1[:, None, None] * g1_ref[0, 0])

    @pl.when(j != 0)
    def _copy():
        out_ref[0, 0] = xr_ref[0, 0]


def _main(x5, n0, n1, s0x, s1x):
    blk = (1, 1, FOLD, H, W)
    grid_spec = pltpu.PrefetchScalarGridSpec(
        num_scalar_prefetch=2,
        grid=(F, NCB),
        in_specs=[
            pl.BlockSpec(blk, lambda i, j, n0, n1: (n0[i], 0, 0, 0, 0)),
            pl.BlockSpec(blk, lambda i, j, n0, n1: (n1[i], 0, 0, 0, 0)),
            pl.BlockSpec(blk, lambda i, j, n0, n1: (i, jnp.maximum(j, 1), 0, 0, 0)),
            pl.BlockSpec((1, 1, FOLD), lambda i, j, n0, n1: (i, 0, 0)),
            pl.BlockSpec((1, 1, FOLD), lambda i, j, n0, n1: (i, 0, 0)),
        ],
        out_specs=pl.BlockSpec(blk, lambda i, j, n0, n1: (i, j, 0, 0, 0)),
    )
    return pl.pallas_call(
        _main_body,
        grid_spec=grid_spec,
        out_shape=jax.ShapeDtypeStruct((F, NCB, FOLD, H, W), jnp.float32),
    )(n0, n1, x5, x5, x5, s0x, s1x)


def kernel(x, off_conv_w, off_conv_b, off_fc1_w, off_fc1_b, off_fc2_w,
           off_fc2_b, w_conv_w, w_conv_b):
    x5 = x.reshape(F, NCB, FOLD, H, W)
    pooled = _pool(x5).reshape(NB, SEG, FOLD)
    n0, n1, s0x, s1x = _nets(
        pooled,
        off_conv_w.reshape(3, FOLD), off_conv_b.reshape(1, 1),
        off_fc1_w, off_fc1_b.reshape(1, SEG),
        off_fc2_w, off_fc2_b.reshape(1, G),
        w_conv_w, w_conv_b.reshape(1, G),
    )
    out = _main(x5, n0.reshape(F), n1.reshape(F),
                s0x.reshape(F, 1, FOLD), s1x.reshape(F, 1, FOLD))
    return out.reshape(F, C, H, W)


# speedup vs baseline: 16.3971x; 8.6996x over previous
import jax
import jax.numpy as jnp
from jax import lax
from jax.experimental import pallas as pl
from jax.experimental.pallas import tpu as pltpu

SEG = 8
NB = 8
F = NB * SEG
C = 512
FOLD = 128
H = 28
W = 28
G = 2
NCB = C // FOLD


def _sigmoid(v):
    return 1.0 / (1.0 + jnp.exp(-v))


def _body(fold_ref, pass_ref, ocw_ref, ocb_ref, f1w_ref, f1b_ref, f2w_ref,
          f2b_ref, wcw_ref, wcb_ref, out_ref, cache_ref, pool_ref, m_ref):
    p = pl.program_id(0)
    b = pl.program_id(1)

    @pl.when(p == 0)
    def _pool_phase():
        blk = fold_ref[...]
        cache_ref[b] = blk
        pool_ref[b] = jnp.sum(blk, axis=(0, 1)) * (1.0 / (H * W))
        out_ref[...] = pass_ref[...]

    @pl.when((p == 1) | (p == 2))
    def _pass_phase():
        out_ref[...] = pass_ref[...]

    @pl.when(p == 3)
    def _blend_phase():
        @pl.when(b == 0)
        def _nets():
            pool = pool_ref[...]
            zero = jnp.zeros((NB, 1, FOLD), jnp.float32)
            p_m1 = jnp.concatenate([zero, pool[:, : SEG - 1]], axis=1)
            p_p1 = jnp.concatenate([pool[:, 1:], zero], axis=1)

            ocw = ocw_ref[...]
            t0 = (jnp.sum(p_m1 * ocw[0][None, None, :], axis=-1)
                  + jnp.sum(pool * ocw[1][None, None, :], axis=-1)
                  + jnp.sum(p_p1 * ocw[2][None, None, :], axis=-1)
                  + ocb_ref[0, 0])
            t1 = jnp.maximum(
                jnp.dot(t0, f1w_ref[...], preferred_element_type=jnp.float32)
                + f1b_ref[0][None, :], 0.0)
            t2 = (jnp.dot(t1, f2w_ref[...], preferred_element_type=jnp.float32)
                  + f2b_ref[0][None, :])
            x_offset = 4.0 * (_sigmoid(t2) - 0.5)
            off_bs = jnp.tile(x_offset, (1, SEG // G))

            off0f = jnp.floor(off_bs)
            o0 = jnp.clip(off0f.astype(jnp.int32), 0, SEG - 1)
            o1 = jnp.clip(off0f.astype(jnp.int32) + 1, 0, SEG - 1)
            w1 = off_bs - off0f
            w0 = 1.0 - w1

            wcw = wcw_ref[...]
            xw = []
            for g in range(G):
                ug = (jnp.sum(p_m1 * wcw[0, :, g][None, None, :], axis=-1)
                      + jnp.sum(pool * wcw[1, :, g][None, None, :], axis=-1)
                      + jnp.sum(p_p1 * wcw[2, :, g][None, None, :], axis=-1)
                      + wcb_ref[0, g])
                xw.append(2.0 * _sigmoid(ug))

            fi = lax.broadcasted_iota(jnp.int32, (NB, SEG, SEG), 2)
            m4 = (jnp.where(fi == o0[:, :, None], w0[:, :, None], 0.0)
                  + jnp.where(fi == o1[:, :, None], w1[:, :, None], 0.0))
            xw_chan = jnp.concatenate(
                [jnp.broadcast_to(xw[0][:, :, None], (NB, SEG, FOLD // G)),
                 jnp.broadcast_to(xw[1][:, :, None], (NB, SEG, FOLD // G))],
                axis=-1)
            m_ref[...] = m4[:, :, :, None] * xw_chan[:, :, None, :]

        m = m_ref[b]
        blk = cache_ref[b]
        acc = blk[:, :, 0, :][:, :, None, :] * m[:, 0, :][None, None, :, :]
        for f in range(1, SEG):
            acc += blk[:, :, f, :][:, :, None, :] * m[:, f, :][None, None, :, :]
        out_ref[...] = acc


def kernel(x, off_conv_w, off_conv_b, off_fc1_w, off_fc1_b, off_fc2_w,
           off_fc2_b, w_conv_w, w_conv_b):
    xt = jnp.transpose(x, (2, 3, 0, 1))
    blk = (H, W, SEG, FOLD)
    out = pl.pallas_call(
        _body,
        grid=(NCB, NB),
        in_specs=[
            pl.BlockSpec(blk, lambda p, b: (0, 0, jnp.where(p == 0, b, 0), 0)),
            pl.BlockSpec(blk, lambda p, b: (0, 0,
                                            jnp.where(p == 3, NB - 1, b),
                                            jnp.where(p == 3, 3, p + 1))),
            pl.BlockSpec((3, FOLD), lambda p, b: (0, 0)),
            pl.BlockSpec((1, 1), lambda p, b: (0, 0)),
            pl.BlockSpec((SEG, SEG), lambda p, b: (0, 0)),
            pl.BlockSpec((1, SEG), lambda p, b: (0, 0)),
            pl.BlockSpec((SEG, G), lambda p, b: (0, 0)),
            pl.BlockSpec((1, G), lambda p, b: (0, 0)),
            pl.BlockSpec((3, FOLD, G), lambda p, b: (0, 0, 0)),
            pl.BlockSpec((1, G), lambda p, b: (0, 0)),
        ],
        out_specs=pl.BlockSpec(blk, lambda p, b: (0, 0, b,
                                                  jnp.where(p == 3, 0, p + 1))),
        out_shape=jax.ShapeDtypeStruct((H, W, F, C), jnp.float32),
        scratch_shapes=[
            pltpu.VMEM((NB, H, W, SEG, FOLD), jnp.float32),
            pltpu.VMEM((NB, SEG, FOLD), jnp.float32),
            pltpu.VMEM((NB, SEG, SEG, FOLD), jnp.float32),
        ],
    )(xt, xt,
      off_conv_w.reshape(3, FOLD), off_conv_b.reshape(1, 1),
      off_fc1_w, off_fc1_b.reshape(1, SEG),
      off_fc2_w, off_fc2_b.reshape(1, G),
      w_conv_w, w_conv_b.reshape(1, G))
    return jnp.transpose(out, (2, 3, 0, 1))
